# baseline (speedup 1.0000x reference)
import functools

import jax
import jax.numpy as jnp
from jax import lax
from jax.experimental import pallas as pl
from jax.experimental.pallas import tpu as pltpu
from jax.experimental.pallas import tpu_sc as plsc

HID = 96
PER = 32
NROWS = 30
NTT = 28672
TBLK = 4096
NW = 32
CH = 128


def _ttb_body(xe_ref, ye_ref, ze_ref, w_ref, b_ref, ttb_ref):
    w = w_ref[...]
    dn = (((1,), (1,)), ((), ()))
    tx = jax.lax.dot_general(xe_ref[...], w[:, 0:32], dn,
                             preferred_element_type=jnp.float32)
    ty = jax.lax.dot_general(ye_ref[...], w[:, 32:64], dn,
                             preferred_element_type=jnp.float32)
    tz = jax.lax.dot_general(ze_ref[...], w[:, 64:96], dn,
                             preferred_element_type=jnp.float32)
    tx = tx + b_ref[...][None, :]
    ttb_ref[...] = jnp.zeros((128, 128), jnp.float32)
    ttb_ref[0:30, 0:HID] = tx
    ttb_ref[32:62, 0:HID] = ty
    ttb_ref[64:94, 0:HID] = tz


def _tt3_body(ttb_ref, tt3_ref):
    base = pl.program_id(0) * TBLK
    c = jax.lax.broadcasted_iota(jnp.int32, (128, TBLK), 1) + base
    ix = jax.lax.shift_right_logical(c * 37283, 25)
    r = c - ix * 900
    iy = jax.lax.shift_right_logical(r * 34953, 20)
    iz = r - iy * 30
    rowc = jax.lax.broadcasted_iota(jnp.int32, (128, TBLK), 0)
    m = (rowc == ix) | (rowc == iy + 32) | (rowc == iz + 64)
    ohT = m.astype(jnp.bfloat16)
    tt3_ref[...] = jax.lax.dot_general(
        ohT, ttb_ref[...].astype(jnp.bfloat16), (((0,), (0,)), ((), ())),
        preferred_element_type=jnp.float32)


def _round_half_even(x):
    t = x + 0.5
    i = t.astype(jnp.int32)
    exact = i.astype(jnp.float32) == t
    odd = (i & 1) == 1
    i = jnp.where(exact & odd, i - 1, i)
    return jnp.minimum(i, 29)


def _sc_body(cx_hbm, cy_hbm, cz_hbm, tt3_hbm, out_hbm,
             cxv, cyv, czv, idxv, rowsv, sem):
    wid = lax.axis_index("s") * 2 + lax.axis_index("c")
    nchunks = (819200 // NW) // CH
    row0 = wid * nchunks

    def chunk(g, carry):
        row = row0 + g
        pltpu.sync_copy(cx_hbm.at[row], cxv)
        pltpu.sync_copy(cy_hbm.at[row], cyv)
        pltpu.sync_copy(cz_hbm.at[row], czv)

        def vstep(i, carry2):
            sl = pl.ds(i * 16, 16)
            ix = _round_half_even(cxv[sl])
            iy = _round_half_even(cyv[sl])
            iz = _round_half_even(czv[sl])
            idxv[sl] = ix * 900 + iy * 30 + iz
            return carry2

        lax.fori_loop(0, CH // 16, vstep, 0)
        pltpu.async_copy(tt3_hbm.at[idxv], rowsv, sem).wait()
        pltpu.sync_copy(rowsv, out_hbm.at[pl.ds(row * CH, CH)])
        return carry

    lax.fori_loop(0, nchunks, chunk, 0)


def kernel(coords, x_emb, y_emb, z_emb, W, b):
    B, S, _ = coords.shape
    n = B * S
    nr = n // 128
    cx = coords[..., 0].reshape(nr, 128)
    cy = coords[..., 1].reshape(nr, 128)
    cz = coords[..., 2].reshape(nr, 128)

    ttb = pl.pallas_call(
        _ttb_body,
        in_specs=[
            pl.BlockSpec((NROWS, PER), lambda: (0, 0)),
            pl.BlockSpec((NROWS, PER), lambda: (0, 0)),
            pl.BlockSpec((NROWS, PER), lambda: (0, 0)),
            pl.BlockSpec((HID, HID), lambda: (0, 0)),
            pl.BlockSpec((HID,), lambda: (0,)),
        ],
        out_specs=pl.BlockSpec((128, 128), lambda: (0, 0)),
        out_shape=jax.ShapeDtypeStruct((128, 128), jnp.float32),
    )(x_emb, y_emb, z_emb, W, b)

    tt3 = pl.pallas_call(
        _tt3_body,
        grid=(NTT // TBLK,),
        in_specs=[pl.BlockSpec((128, 128), lambda i: (0, 0))],
        out_specs=pl.BlockSpec((TBLK, 128), lambda i: (i, 0)),
        out_shape=jax.ShapeDtypeStruct((NTT, 128), jnp.float32),
    )(ttb)

    mesh = plsc.VectorSubcoreMesh(core_axis_name="c", subcore_axis_name="s")
    out128 = pl.kernel(
        _sc_body,
        mesh=mesh,
        out_type=jax.ShapeDtypeStruct((n, 128), jnp.float32),
        scratch_types=[
            pltpu.VMEM((CH,), jnp.float32),
            pltpu.VMEM((CH,), jnp.float32),
            pltpu.VMEM((CH,), jnp.float32),
            pltpu.VMEM((CH,), jnp.int32),
            pltpu.VMEM((CH, 128), jnp.float32),
            pltpu.SemaphoreType.DMA,
        ],
    )(cx, cy, cz, tt3)
    return out128[:, :HID].reshape(B, S, HID)


# speedup vs baseline: 2.2528x; 2.2528x over previous
import jax
import jax.numpy as jnp
from jax.experimental import pallas as pl
from jax.experimental.pallas import tpu as pltpu

HID = 96
PER = 32
NROWS = 30
BLK = 40960
SUBB = BLK // 128


def _tc_body(cx_ref, cy_ref, cz_ref, xe_ref, ye_ref, ze_ref, w_ref, b_ref,
             out_ref, ttb_ref):
    @pl.when(pl.program_id(0) == 0)
    def _init():
        w = w_ref[...]
        dn = (((1,), (1,)), ((), ()))
        tx = jax.lax.dot_general(xe_ref[...], w[:, 0:32], dn,
                                 preferred_element_type=jnp.float32)
        ty = jax.lax.dot_general(ye_ref[...], w[:, 32:64], dn,
                                 preferred_element_type=jnp.float32)
        tz = jax.lax.dot_general(ze_ref[...], w[:, 64:96], dn,
                                 preferred_element_type=jnp.float32)
        tx = tx + b_ref[...][None, :]
        ttb_ref[...] = jnp.zeros((128, HID), jnp.bfloat16)
        ttb_ref[0:30, :] = tx.astype(jnp.bfloat16)
        ttb_ref[30:31, :] = tx[29:30, :].astype(jnp.bfloat16)
        ttb_ref[32:62, :] = ty.astype(jnp.bfloat16)
        ttb_ref[62:63, :] = ty[29:30, :].astype(jnp.bfloat16)
        ttb_ref[64:94, :] = tz.astype(jnp.bfloat16)
        ttb_ref[94:95, :] = tz[29:30, :].astype(jnp.bfloat16)

    rx = jnp.round(cx_ref[...])
    ry = jnp.round(cy_ref[...]) + 32.0
    rz = jnp.round(cz_ref[...]) + 64.0
    rowc = jax.lax.broadcasted_iota(jnp.int32, (128, 128), 0).astype(
        jnp.float32)
    chunks = []
    for j in range(SUBB):
        m = ((rowc == rx[j:j + 1, :]) | (rowc == ry[j:j + 1, :])
             | (rowc == rz[j:j + 1, :]))
        chunks.append(m)
    ohT = jnp.concatenate(chunks, axis=1).astype(jnp.bfloat16)
    out_ref[...] = jax.lax.dot_general(
        ohT, ttb_ref[...], (((0,), (0,)), ((), ())),
        preferred_element_type=jnp.float32)


def kernel(coords, x_emb, y_emb, z_emb, W, b):
    B, S, _ = coords.shape
    n = B * S
    nr = n // 128
    cx = coords[..., 0].reshape(nr, 128)
    cy = coords[..., 1].reshape(nr, 128)
    cz = coords[..., 2].reshape(nr, 128)
    cspec = pl.BlockSpec((SUBB, 128), lambda i: (i, 0))
    out = pl.pallas_call(
        _tc_body,
        grid=(n // BLK,),
        in_specs=[
            cspec, cspec, cspec,
            pl.BlockSpec((NROWS, PER), lambda i: (0, 0)),
            pl.BlockSpec((NROWS, PER), lambda i: (0, 0)),
            pl.BlockSpec((NROWS, PER), lambda i: (0, 0)),
            pl.BlockSpec((HID, HID), lambda i: (0, 0)),
            pl.BlockSpec((HID,), lambda i: (0,)),
        ],
        out_specs=pl.BlockSpec((BLK, HID), lambda i: (i, 0)),
        out_shape=jax.ShapeDtypeStruct((n, HID), jnp.float32),
        scratch_shapes=[pltpu.VMEM((128, HID), jnp.bfloat16)],
    )(cx, cy, cz, x_emb, y_emb, z_emb, W, b)
    return out.reshape(B, S, HID)
